# baseline (speedup 1.0000x reference)
import functools

import jax
import jax.numpy as jnp
from jax import lax
from jax.experimental import pallas as pl
from jax.experimental.pallas import tpu as pltpu
from jax.experimental.pallas import tpu_sc as plsc

_NC = 2
_NS = 16
_L = 16
_C = 10000
_NBUF = 2
_U = 25


@functools.lru_cache(maxsize=None)
def _make_segsum(E_pad: int, N: int, DE: int):
    assert DE == _NS, "one feature column per subcore"
    e_half = E_pad // _NC
    n_chunks = e_half // _C
    n_groups = _C // _L
    assert n_chunks % _NBUF == 0 and n_groups % _U == 0
    mesh = plsc.VectorSubcoreMesh(
        core_axis_name="c", subcore_axis_name="s",
        num_cores=_NC, num_subcores=_NS)

    @functools.partial(
        pl.kernel,
        out_type=jax.ShapeDtypeStruct((_NC * DE * N,), jnp.float32),
        mesh=mesh,
        compiler_params=pltpu.CompilerParams(needs_layout_passes=False),
        scratch_types=[
            pltpu.VMEM((_C,), jnp.float32),
            pltpu.VMEM((_C,), jnp.float32),
            pltpu.VMEM((_C,), jnp.int32),
            pltpu.VMEM((_C,), jnp.int32),
            pltpu.VMEM((N,), jnp.float32),
            pltpu.SemaphoreType.DMA((_NBUF,)),
            pltpu.SemaphoreType.DMA((_NBUF,)),
        ],
    )
    def segsum(vals_hbm, idx_hbm, out_hbm, vals_v0, vals_v1, idx_v0, idx_v1,
               acc, vsem, isem):
        vals_bufs = (vals_v0, vals_v1)
        idx_bufs = (idx_v0, idx_v1)
        c = lax.axis_index("c")
        s = lax.axis_index("s")

        def zbody(i, carry):
            acc[pl.ds(i * _L, _L)] = jnp.zeros((_L,), jnp.float32)
            return carry

        lax.fori_loop(0, N // _L, zbody, 0)

        vbase = s * E_pad + c * e_half
        ibase = c * e_half

        def fetch(t, b):
            pltpu.async_copy(vals_hbm.at[pl.ds(vbase + t * _C, _C)],
                             vals_bufs[b], vsem.at[b])
            pltpu.async_copy(idx_hbm.at[pl.ds(ibase + t * _C, _C)],
                             idx_bufs[b], isem.at[b])

        for b in range(_NBUF):
            fetch(b, b)

        def outer(tt, carry):
            for b in range(_NBUF):
                t = tt * _NBUF + b
                pltpu.make_async_copy(vals_hbm.at[pl.ds(vbase, _C)],
                                      vals_bufs[b], vsem.at[b]).wait()
                pltpu.make_async_copy(idx_hbm.at[pl.ds(ibase, _C)],
                                      idx_bufs[b], isem.at[b]).wait()

                @plsc.parallel_loop(0, n_groups, unroll=_U)
                def gbody(g):
                    off = g * _L
                    iv = idx_bufs[b][pl.ds(off, _L)]
                    vv = vals_bufs[b][pl.ds(off, _L)]
                    plsc.addupdate_scatter(acc, [iv], vv)

                @pl.when(t + _NBUF < n_chunks)
                def _():
                    fetch(t + _NBUF, b)
            return carry

        lax.fori_loop(0, n_chunks // _NBUF, outer, 0)
        pltpu.sync_copy(acc, out_hbm.at[pl.ds((c * DE + s) * N, N)])

    return segsum


@functools.lru_cache(maxsize=None)
def _make_transpose(E_pad: int, DE: int, BE: int):
    def body(x_r, eye_r, o_r):
        o_r[...] = jax.lax.dot_general(
            eye_r[...], x_r[...], (((1,), (1,)), ((), ())),
            preferred_element_type=jnp.float32)

    return pl.pallas_call(
        body,
        grid=(E_pad // BE,),
        in_specs=[
            pl.BlockSpec((BE, DE), lambda i: (i, 0)),
            pl.BlockSpec((DE, DE), lambda i: (0, 0)),
        ],
        out_specs=pl.BlockSpec((DE, BE), lambda i: (0, i)),
        out_shape=jax.ShapeDtypeStruct((DE, E_pad), jnp.float32),
    )


@functools.lru_cache(maxsize=None)
def _make_mlp(N: int, DF: int, DE: int, H: int, BN: int):
    def body(x_r, p_r, w1x_r, w1e_r, b1_r, w2_r, b2_r, o_r):
        p = p_r[...]
        aggT = p[:DE] + p[DE:]
        h = jnp.dot(x_r[...], w1x_r[...], preferred_element_type=jnp.float32)
        h += jax.lax.dot_general(aggT, w1e_r[...],
                                 (((0,), (0,)), ((), ())),
                                 preferred_element_type=jnp.float32)
        h = jnp.maximum(h + b1_r[...], 0.0)
        o_r[...] = jnp.dot(h, w2_r[...],
                           preferred_element_type=jnp.float32) + b2_r[...]

    return pl.pallas_call(
        body,
        grid=(pl.cdiv(N, BN),),
        in_specs=[
            pl.BlockSpec((BN, DF), lambda i: (i, 0)),
            pl.BlockSpec((_NC * DE, BN), lambda i: (0, i)),
            pl.BlockSpec((DF, H), lambda i: (0, 0)),
            pl.BlockSpec((DE, H), lambda i: (0, 0)),
            pl.BlockSpec((1, H), lambda i: (0, 0)),
            pl.BlockSpec((H, H), lambda i: (0, 0)),
            pl.BlockSpec((1, H), lambda i: (0, 0)),
        ],
        out_specs=pl.BlockSpec((BN, H), lambda i: (i, 0)),
        out_shape=jax.ShapeDtypeStruct((N, H), jnp.float32),
    )


def kernel(x, edge_attr, edge_index, W1, b1, W2, b2):
    N, DF = x.shape
    E, DE = edge_attr.shape
    H = W2.shape[1]
    recv = edge_index[1]

    chunk = 160000
    E_pad = ((E + chunk - 1) // chunk) * chunk
    if E_pad != E:
        pad = E_pad - E
        edge_in = jnp.concatenate(
            [edge_attr, jnp.zeros((pad, DE), jnp.float32)], axis=0)
        recv = jnp.concatenate([recv, jnp.zeros((pad,), jnp.int32)], axis=0)
    else:
        edge_in = edge_attr
    BE = 32000
    eye = jnp.eye(DE, dtype=jnp.float32)
    vals_flat = _make_transpose(E_pad, DE, BE)(edge_in, eye).reshape(-1)

    parts = _make_segsum(E_pad, N, DE)(vals_flat, recv)
    parts2 = parts.reshape(_NC * DE, N)

    BN = 1024
    out = _make_mlp(N, DF, DE, H, BN)(
        x, parts2, W1[:DF], W1[DF:], b1.reshape(1, H), W2, b2.reshape(1, H))
    return (out, edge_attr, edge_index)


# speedup vs baseline: 2.2051x; 2.2051x over previous
import functools

import jax
import jax.numpy as jnp
from jax import lax
from jax.experimental import pallas as pl
from jax.experimental.pallas import tpu as pltpu
from jax.experimental.pallas import tpu_sc as plsc

_NC = 2
_NS = 16
_L = 16
_C = 8000
_NBUF = 2
_U = 25


@functools.lru_cache(maxsize=None)
def _make_segsum(E_pad: int, N: int, DE: int):
    assert DE == _NS, "one feature column per subcore"
    e_half = E_pad // _NC
    n_chunks = e_half // _C
    n_groups = _C // _L
    assert n_chunks % _NBUF == 0 and n_groups % _U == 0
    mesh = plsc.VectorSubcoreMesh(
        core_axis_name="c", subcore_axis_name="s",
        num_cores=_NC, num_subcores=_NS)

    @functools.partial(
        pl.kernel,
        out_type=jax.ShapeDtypeStruct((_NC * DE * N,), jnp.float32),
        mesh=mesh,
        compiler_params=pltpu.CompilerParams(needs_layout_passes=False),
        scratch_types=[
            pltpu.VMEM((_C,), jnp.float32),
            pltpu.VMEM((_C,), jnp.float32),
            pltpu.VMEM((_C,), jnp.int32),
            pltpu.VMEM((_C,), jnp.int32),
            pltpu.VMEM((N,), jnp.float32),
            pltpu.VMEM((N,), jnp.float32),
            pltpu.SemaphoreType.DMA((_NBUF,)),
            pltpu.SemaphoreType.DMA((_NBUF,)),
        ],
    )
    def segsum(vals_hbm, idx_hbm, out_hbm, vals_v0, vals_v1, idx_v0, idx_v1,
               acc, acc2, vsem, isem):
        vals_bufs = (vals_v0, vals_v1)
        idx_bufs = (idx_v0, idx_v1)
        c = lax.axis_index("c")
        s = lax.axis_index("s")

        def zbody(i, carry):
            acc[pl.ds(i * _L, _L)] = jnp.zeros((_L,), jnp.float32)
            acc2[pl.ds(i * _L, _L)] = jnp.zeros((_L,), jnp.float32)
            return carry

        lax.fori_loop(0, N // _L, zbody, 0)

        vbase = s * E_pad + c * e_half
        ibase = c * e_half

        def fetch(t, b):
            pltpu.async_copy(vals_hbm.at[pl.ds(vbase + t * _C, _C)],
                             vals_bufs[b], vsem.at[b])
            pltpu.async_copy(idx_hbm.at[pl.ds(ibase + t * _C, _C)],
                             idx_bufs[b], isem.at[b])

        for b in range(_NBUF):
            fetch(b, b)

        def outer(tt, carry):
            for b in range(_NBUF):
                t = tt * _NBUF + b
                pltpu.make_async_copy(vals_hbm.at[pl.ds(vbase, _C)],
                                      vals_bufs[b], vsem.at[b]).wait()
                pltpu.make_async_copy(idx_hbm.at[pl.ds(ibase, _C)],
                                      idx_bufs[b], isem.at[b]).wait()

                @plsc.parallel_loop(0, n_groups // 2, unroll=_U)
                def gbody(g):
                    off = g * (2 * _L)
                    iv = idx_bufs[b][pl.ds(off, _L)]
                    vv = vals_bufs[b][pl.ds(off, _L)]
                    plsc.addupdate_scatter(acc, [iv], vv)
                    iv2 = idx_bufs[b][pl.ds(off + _L, _L)]
                    vv2 = vals_bufs[b][pl.ds(off + _L, _L)]
                    plsc.addupdate_scatter(acc2, [iv2], vv2)

                @pl.when(t + _NBUF < n_chunks)
                def _():
                    fetch(t + _NBUF, b)
            return carry

        lax.fori_loop(0, n_chunks // _NBUF, outer, 0)

        @plsc.parallel_loop(0, N // _L, unroll=_U)
        def mbody(i):
            off = i * _L
            acc[pl.ds(off, _L)] = acc[pl.ds(off, _L)] + acc2[pl.ds(off, _L)]

        pltpu.sync_copy(acc, out_hbm.at[pl.ds((c * DE + s) * N, N)])

    return segsum


@functools.lru_cache(maxsize=None)
def _make_mlp(N: int, DF: int, DE: int, H: int, BN: int):
    def body(x_r, p_r, w1x_r, w1e_r, b1_r, w2_r, b2_r, o_r):
        p = p_r[...]
        aggT = p[:DE] + p[DE:]
        h = jnp.dot(x_r[...], w1x_r[...], preferred_element_type=jnp.float32)
        h += jax.lax.dot_general(aggT, w1e_r[...],
                                 (((0,), (0,)), ((), ())),
                                 preferred_element_type=jnp.float32)
        h = jnp.maximum(h + b1_r[...], 0.0)
        o_r[...] = jnp.dot(h, w2_r[...],
                           preferred_element_type=jnp.float32) + b2_r[...]

    return pl.pallas_call(
        body,
        grid=(pl.cdiv(N, BN),),
        in_specs=[
            pl.BlockSpec((BN, DF), lambda i: (i, 0)),
            pl.BlockSpec((_NC * DE, BN), lambda i: (0, i)),
            pl.BlockSpec((DF, H), lambda i: (0, 0)),
            pl.BlockSpec((DE, H), lambda i: (0, 0)),
            pl.BlockSpec((1, H), lambda i: (0, 0)),
            pl.BlockSpec((H, H), lambda i: (0, 0)),
            pl.BlockSpec((1, H), lambda i: (0, 0)),
        ],
        out_specs=pl.BlockSpec((BN, H), lambda i: (i, 0)),
        out_shape=jax.ShapeDtypeStruct((N, H), jnp.float32),
    )


def kernel(x, edge_attr, edge_index, W1, b1, W2, b2):
    N, DF = x.shape
    E, DE = edge_attr.shape
    H = W2.shape[1]
    recv = edge_index[1]

    chunk = _NC * _C * _NBUF
    E_pad = ((E + chunk - 1) // chunk) * chunk
    if E_pad != E:
        pad = E_pad - E
        edge_in = jnp.concatenate(
            [edge_attr, jnp.zeros((pad, DE), jnp.float32)], axis=0)
        recv = jnp.concatenate([recv, jnp.zeros((pad,), jnp.int32)], axis=0)
    else:
        edge_in = edge_attr
    vals_flat = edge_in.T.reshape(-1)

    parts = _make_segsum(E_pad, N, DE)(vals_flat, recv)
    parts2 = parts.reshape(_NC * DE, N)

    BN = 1024
    out = _make_mlp(N, DF, DE, H, BN)(
        x, parts2, W1[:DF], W1[DF:], b1.reshape(1, H), W2, b2.reshape(1, H))
    return (out, edge_attr, edge_index)


# speedup vs baseline: 2.2307x; 1.0116x over previous
import functools

import jax
import jax.numpy as jnp
from jax import lax
from jax.experimental import pallas as pl
from jax.experimental.pallas import tpu as pltpu
from jax.experimental.pallas import tpu_sc as plsc

_NC = 2
_NS = 16
_L = 16
_C = 10000
_NBUF = 2
_U = 25


@functools.lru_cache(maxsize=None)
def _make_segsum(E_pad: int, N: int, DE: int):
    assert DE == _NS, "one feature column per subcore"
    e_half = E_pad // _NC
    n_chunks = e_half // _C
    n_groups = _C // _L
    assert n_chunks % _NBUF == 0 and n_groups % _U == 0
    mesh = plsc.VectorSubcoreMesh(
        core_axis_name="c", subcore_axis_name="s",
        num_cores=_NC, num_subcores=_NS)

    @functools.partial(
        pl.kernel,
        out_type=jax.ShapeDtypeStruct((_NC * DE * N,), jnp.float32),
        mesh=mesh,
        compiler_params=pltpu.CompilerParams(needs_layout_passes=False),
        scratch_types=[
            pltpu.VMEM((_C,), jnp.float32),
            pltpu.VMEM((_C,), jnp.float32),
            pltpu.VMEM((_C,), jnp.int32),
            pltpu.VMEM((_C,), jnp.int32),
            pltpu.VMEM((N,), jnp.float32),
            pltpu.SemaphoreType.DMA((_NBUF,)),
            pltpu.SemaphoreType.DMA((_NBUF,)),
        ],
    )
    def segsum(vals_hbm, idx_hbm, out_hbm, vals_v0, vals_v1, idx_v0, idx_v1,
               acc, vsem, isem):
        vals_bufs = (vals_v0, vals_v1)
        idx_bufs = (idx_v0, idx_v1)
        c = lax.axis_index("c")
        s = lax.axis_index("s")

        def zbody(i, carry):
            acc[pl.ds(i * _L, _L)] = jnp.zeros((_L,), jnp.float32)
            return carry

        lax.fori_loop(0, N // _L, zbody, 0)

        vbase = s * E_pad + c * e_half
        ibase = c * e_half

        def fetch(t, b):
            pltpu.async_copy(vals_hbm.at[pl.ds(vbase + t * _C, _C)],
                             vals_bufs[b], vsem.at[b])
            pltpu.async_copy(idx_hbm.at[pl.ds(ibase + t * _C, _C)],
                             idx_bufs[b], isem.at[b])

        for b in range(_NBUF):
            fetch(b, b)

        def outer(tt, carry):
            for b in range(_NBUF):
                t = tt * _NBUF + b
                pltpu.make_async_copy(vals_hbm.at[pl.ds(vbase, _C)],
                                      vals_bufs[b], vsem.at[b]).wait()
                pltpu.make_async_copy(idx_hbm.at[pl.ds(ibase, _C)],
                                      idx_bufs[b], isem.at[b]).wait()

                @plsc.parallel_loop(0, n_groups, unroll=_U)
                def gbody(g):
                    off = g * _L
                    iv = idx_bufs[b][pl.ds(off, _L)]
                    vv = vals_bufs[b][pl.ds(off, _L)]
                    plsc.addupdate_scatter(acc, [iv], vv)

                @pl.when(t + _NBUF < n_chunks)
                def _():
                    fetch(t + _NBUF, b)
            return carry

        lax.fori_loop(0, n_chunks // _NBUF, outer, 0)
        pltpu.sync_copy(acc, out_hbm.at[pl.ds((c * DE + s) * N, N)])

    return segsum


@functools.lru_cache(maxsize=None)
def _make_mlp(N: int, DF: int, DE: int, H: int, BN: int):
    def body(x_r, p_r, w1x_r, w1e_r, b1_r, w2_r, b2_r, o_r):
        p = p_r[...]
        aggT = p[:DE] + p[DE:]
        h = jnp.dot(x_r[...], w1x_r[...], preferred_element_type=jnp.float32)
        h += jax.lax.dot_general(aggT, w1e_r[...],
                                 (((0,), (0,)), ((), ())),
                                 preferred_element_type=jnp.float32)
        h = jnp.maximum(h + b1_r[...], 0.0)
        o_r[...] = jnp.dot(h, w2_r[...],
                           preferred_element_type=jnp.float32) + b2_r[...]

    return pl.pallas_call(
        body,
        grid=(pl.cdiv(N, BN),),
        in_specs=[
            pl.BlockSpec((BN, DF), lambda i: (i, 0)),
            pl.BlockSpec((_NC * DE, BN), lambda i: (0, i)),
            pl.BlockSpec((DF, H), lambda i: (0, 0)),
            pl.BlockSpec((DE, H), lambda i: (0, 0)),
            pl.BlockSpec((1, H), lambda i: (0, 0)),
            pl.BlockSpec((H, H), lambda i: (0, 0)),
            pl.BlockSpec((1, H), lambda i: (0, 0)),
        ],
        out_specs=pl.BlockSpec((BN, H), lambda i: (i, 0)),
        out_shape=jax.ShapeDtypeStruct((N, H), jnp.float32),
    )


def kernel(x, edge_attr, edge_index, W1, b1, W2, b2):
    N, DF = x.shape
    E, DE = edge_attr.shape
    H = W2.shape[1]
    recv = edge_index[1]

    chunk = _NC * _C * _NBUF
    E_pad = ((E + chunk - 1) // chunk) * chunk
    if E_pad != E:
        pad = E_pad - E
        edge_in = jnp.concatenate(
            [edge_attr, jnp.zeros((pad, DE), jnp.float32)], axis=0)
        recv = jnp.concatenate([recv, jnp.zeros((pad,), jnp.int32)], axis=0)
    else:
        edge_in = edge_attr
    vals_flat = edge_in.T.reshape(-1)

    parts = _make_segsum(E_pad, N, DE)(vals_flat, recv)
    parts2 = parts.reshape(_NC * DE, N)

    BN = 1024
    out = _make_mlp(N, DF, DE, H, BN)(
        x, parts2, W1[:DF], W1[DF:], b1.reshape(1, H), W2, b2.reshape(1, H))
    return (out, edge_attr, edge_index)
